# baseline (speedup 1.0000x reference)
import functools

import jax
import jax.numpy as jnp
from jax import lax
from jax.experimental import pallas as pl
from jax.experimental.pallas import tpu as pltpu
from jax.experimental.pallas import tpu_sc as plsc

_VOCAB = 256000
_HIDDEN = 2560
_NC = 2
_NS = 16
_NW = _NC * _NS
_B = 4 * 4096
_BPW = _B // _NW
_CH = 8
_NCH = _BPW // _CH
_NBUF = 4


@functools.partial(
    pl.kernel,
    out_type=jax.ShapeDtypeStruct((_B, _HIDDEN), jnp.float32),
    mesh=plsc.VectorSubcoreMesh(core_axis_name="c", subcore_axis_name="s"),
    scratch_types=(
        [pltpu.VMEM((_BPW,), jnp.int32)]
        + [pltpu.VMEM((_CH, _HIDDEN), jnp.float32)] * _NBUF
        + [pltpu.SemaphoreType.DMA] * _NBUF
    ),
)
def _emb_lookup(x_hbm, table_hbm, out_hbm, idx_v, *bufsems):
    bufs = bufsems[:_NBUF]
    sems = bufsems[_NBUF:]
    wid = lax.axis_index("s") * _NC + lax.axis_index("c")
    base = wid * _BPW
    pltpu.sync_copy(x_hbm.at[pl.ds(base, _BPW)], idx_v)

    def start(c, b):
        pltpu.async_copy(table_hbm.at[idx_v.at[pl.ds(c * _CH, _CH)]],
                         bufs[b], sems[b])

    def wait(b):
        pltpu.make_async_copy(table_hbm.at[pl.ds(0, _CH)], bufs[b],
                              sems[b]).wait()

    for b in range(_NBUF):
        start(b, b)

    def body(i, carry):
        c0 = i * _NBUF
        for b in range(_NBUF):
            c = c0 + b
            wait(b)
            pltpu.sync_copy(bufs[b], out_hbm.at[pl.ds(base + c * _CH, _CH)])
            start(c + _NBUF, b)
        return carry

    lax.fori_loop(0, _NCH // _NBUF - 1, body, 0)

    for b in range(_NBUF):
        c = _NCH - _NBUF + b
        wait(b)
        pltpu.sync_copy(bufs[b], out_hbm.at[pl.ds(base + c * _CH, _CH)])


def kernel(x, table):
    idx = jnp.clip(x.reshape(-1).astype(jnp.int32), 0, table.shape[0] - 1)
    out = _emb_lookup(idx, table)
    return out.reshape(x.shape + (table.shape[1],))
